# baseline (speedup 1.0000x reference)
import functools

import jax
import jax.numpy as jnp
from jax import lax
from jax.experimental import pallas as pl
from jax.experimental.pallas import tpu as pltpu
from jax.experimental.pallas import tpu_sc as plsc

B = 16384
D = 128
V = 1000000
NC = 2
NS = 16
NW = NC * NS
BPW = B // NW
CHUNK = 128
NCH = BPW // CHUNK
CHR = 128
NCHR = BPW // CHR

_sc_mesh = plsc.VectorSubcoreMesh(core_axis_name="c", subcore_axis_name="s")


@functools.partial(
    pl.kernel,
    out_type=jax.ShapeDtypeStruct((4, B), jnp.float32),
    mesh=_sc_mesh,
    scratch_types=[
        pltpu.VMEM((NCH, CHUNK), jnp.int32),
        pltpu.VMEM((NCH, CHUNK), jnp.int32),
        pltpu.VMEM((NCH, CHUNK, 16), jnp.float32),
        pltpu.VMEM((2, CHR, D), jnp.float32),
        pltpu.VMEM((1, D), jnp.float32),
        pltpu.VMEM((1, 16), jnp.float32),
        pltpu.VMEM((16,), jnp.float32),
        pltpu.VMEM((4, BPW), jnp.float32),
        pltpu.SemaphoreType.DMA,
        pltpu.SemaphoreType.DMA,
    ],
    compiler_params=pltpu.CompilerParams(
        needs_layout_passes=False, use_tc_tiling_on_sc=False),
)
def _sc_fused(fv_hbm, idx_hbm, table_hbm, w_hbm, f_hbm, b_hbm, out_hbm,
              idx_v, row_v, e_v, fv_v, w_v, f_v, b_v, out_v, gsem, fsem):
    wid = lax.axis_index("s") * NC + lax.axis_index("c")
    base = wid * BPW
    cps = [None, None]
    cps[0] = pltpu.async_copy(fv_hbm.at[pl.ds(base, CHR)], fv_v.at[0], fsem)
    pltpu.sync_copy(idx_hbm.at[wid], idx_v)
    table16 = table_hbm
    for j in range(NCH):
        for t in range(CHUNK // 16):
            row_v[j, pl.ds(t * 16, 16)] = idx_v[j, pl.ds(t * 16, 16)] >> 4
    gathers = [
        pltpu.async_copy(table16.at[row_v.at[j]], e_v.at[j], gsem)
        for j in range(NCH)
    ]
    pltpu.sync_copy(w_hbm, w_v)
    pltpu.sync_copy(f_hbm, f_v.at[pl.ds(0, 1), pl.ds(0, 1)])
    pltpu.sync_copy(b_hbm, b_v.at[pl.ds(0, 1)])
    lanes = lax.iota(jnp.int32, 16)
    ones16 = jnp.full((16,), 1.0, jnp.float32)
    fvec = ones16 * f_v[0, pl.ds(0, 16)][0]
    bias = ones16 * b_v[pl.ds(0, 16)][0]

    for k in range(NCHR):
        if k + 1 < NCHR:
            cps[(k + 1) % 2] = pltpu.async_copy(
                fv_hbm.at[pl.ds(base + (k + 1) * CHR, CHR)], fv_v.at[(k + 1) % 2], fsem)
        cps[k % 2].wait()
        fvk = fv_v.at[k % 2]

        def group(g, carry):
            r0 = g * 16
            p0 = k * CHR + r0
            out_v[0, pl.ds(p0, 16)] = ones16
            out_v[1, pl.ds(p0, 16)] = fvec
            dots = jnp.zeros((16,), jnp.float32)
            for r in range(16):
                acc = fvk[r0 + r, pl.ds(0, 16)] * w_v[0, pl.ds(0, 16)]
                for c in range(1, 8):
                    acc = acc + fvk[r0 + r, pl.ds(c * 16, 16)] * w_v[0, pl.ds(c * 16, 16)]
                dots = jnp.where(lanes == r, jnp.sum(acc), dots)
            out_v[3, pl.ds(p0, 16)] = dots + bias
            return carry

        lax.fori_loop(0, CHR // 16, group, 0)

    for c in gathers:
        c.wait()

    def egroup(g, carry):
        rows = g * 16 + lanes
        jv = rows >> 7
        kv = rows & 127
        ids = plsc.load_gather(idx_v, [jv, kv])
        ev = plsc.load_gather(e_v, [jv, kv, ids & 15])
        out_v[2, pl.ds(g * 16, 16)] = ev
        return carry

    lax.fori_loop(0, BPW // 16, egroup, 0)
    pltpu.sync_copy(out_v, out_hbm.at[:, pl.ds(base, BPW)])


def kernel(fixed_vectors, item_id, F_param, emb_table, T_weight, T_bias):
    idx = jnp.asarray(item_id, jnp.int32).reshape(NW, NCH, CHUNK)
    tbl16 = jnp.pad(emb_table, ((0, 64), (0, 0))).reshape((V + 64) // 16, 16)
    out_t = _sc_fused(fixed_vectors, idx, tbl16, T_weight, F_param, T_bias)
    return out_t.T


# speedup vs baseline: 1.0865x; 1.0865x over previous
import functools

import jax
import jax.numpy as jnp
from jax import lax
from jax.experimental import pallas as pl
from jax.experimental.pallas import tpu as pltpu
from jax.experimental.pallas import tpu_sc as plsc

B = 16384
D = 128
V = 1000000
NC = 2
NS = 16
NW = NC * NS
BPW = B // NW
CHUNK = 128
NCH = BPW // CHUNK
CHR = 128
NCHR = BPW // CHR

_sc_mesh = plsc.VectorSubcoreMesh(core_axis_name="c", subcore_axis_name="s")


@functools.partial(
    pl.kernel,
    out_type=jax.ShapeDtypeStruct((4, B), jnp.float32),
    mesh=_sc_mesh,
    scratch_types=[
        pltpu.VMEM((NCH, CHUNK), jnp.int32),
        pltpu.VMEM((NCH, CHUNK), jnp.float32),
        pltpu.VMEM((2, CHR, D), jnp.float32),
        pltpu.VMEM((1, D), jnp.float32),
        pltpu.VMEM((1, 16), jnp.float32),
        pltpu.VMEM((16,), jnp.float32),
        pltpu.VMEM((4, BPW), jnp.float32),
        pltpu.SemaphoreType.DMA,
        pltpu.SemaphoreType.DMA,
    ],
    compiler_params=pltpu.CompilerParams(
        needs_layout_passes=False, use_tc_tiling_on_sc=False),
)
def _sc_fused(fv_hbm, idx_hbm, table_hbm, w_hbm, f_hbm, b_hbm, out_hbm,
              idx_v, e_v, fv_v, w_v, f_v, b_v, out_v, gsem, fsem):
    wid = lax.axis_index("s") * NC + lax.axis_index("c")
    base = wid * BPW
    cps = [None, None]
    cps[0] = pltpu.async_copy(fv_hbm.at[pl.ds(base, CHR)], fv_v.at[0], fsem)
    pltpu.sync_copy(idx_hbm.at[wid], idx_v)
    tflat = table_hbm.at[0]
    gathers = [
        pltpu.async_copy(tflat.at[idx_v.at[j]], e_v.at[j], gsem)
        for j in range(NCH)
    ]
    pltpu.sync_copy(w_hbm, w_v)
    pltpu.sync_copy(f_hbm, f_v.at[pl.ds(0, 1), pl.ds(0, 1)])
    pltpu.sync_copy(b_hbm, b_v.at[pl.ds(0, 1)])
    lanes = lax.iota(jnp.int32, 16)
    ones16 = jnp.full((16,), 1.0, jnp.float32)
    fvec = ones16 * f_v[0, pl.ds(0, 16)][0]
    bias = ones16 * b_v[pl.ds(0, 16)][0]

    for k in range(NCHR):
        if k + 1 < NCHR:
            cps[(k + 1) % 2] = pltpu.async_copy(
                fv_hbm.at[pl.ds(base + (k + 1) * CHR, CHR)], fv_v.at[(k + 1) % 2], fsem)
        cps[k % 2].wait()
        fvk = fv_v.at[k % 2]

        def group(g, carry):
            r0 = g * 16
            p0 = k * CHR + r0
            out_v[0, pl.ds(p0, 16)] = ones16
            out_v[1, pl.ds(p0, 16)] = fvec
            dots = jnp.zeros((16,), jnp.float32)
            for r in range(16):
                acc = fvk[r0 + r, pl.ds(0, 16)] * w_v[0, pl.ds(0, 16)]
                for c in range(1, 8):
                    acc = acc + fvk[r0 + r, pl.ds(c * 16, 16)] * w_v[0, pl.ds(c * 16, 16)]
                dots = jnp.where(lanes == r, jnp.sum(acc), dots)
            out_v[3, pl.ds(p0, 16)] = dots + bias
            return carry

        lax.fori_loop(0, CHR // 16, group, 0)

    for c in gathers:
        c.wait()

    def egroup(g, carry):
        rows = g * 16 + lanes
        ev = plsc.load_gather(e_v, [rows >> 7, rows & 127])
        out_v[2, pl.ds(g * 16, 16)] = ev
        return carry

    lax.fori_loop(0, BPW // 16, egroup, 0)
    pltpu.sync_copy(out_v, out_hbm.at[:, pl.ds(base, BPW)])


def kernel(fixed_vectors, item_id, F_param, emb_table, T_weight, T_bias):
    idx = jnp.asarray(item_id, jnp.int32).reshape(NW, NCH, CHUNK)
    out_t = _sc_fused(fixed_vectors, idx, emb_table.T, T_weight, F_param, T_bias)
    return out_t.T


# speedup vs baseline: 1.1980x; 1.1026x over previous
import functools

import jax
import jax.numpy as jnp
from jax import lax
from jax.experimental import pallas as pl
from jax.experimental.pallas import tpu as pltpu
from jax.experimental.pallas import tpu_sc as plsc

B = 16384
D = 128
V = 1000000
NC = 2
NS = 16
NW = NC * NS
BPW = B // NW
CHUNK = 128
NCH = BPW // CHUNK
CHR = 128
NCHR = BPW // CHR
VPW = 31248
VTAIL = V - NW * VPW

_sc_mesh = plsc.VectorSubcoreMesh(core_axis_name="c", subcore_axis_name="s")


@functools.partial(
    pl.kernel,
    out_type=jax.ShapeDtypeStruct((4, B), jnp.float32),
    mesh=_sc_mesh,
    scratch_types=[
        pltpu.VMEM((2, CHR, D), jnp.float32),
        pltpu.VMEM((1, D), jnp.float32),
        pltpu.VMEM((1, 16), jnp.float32),
        pltpu.VMEM((16,), jnp.float32),
        pltpu.VMEM((4, BPW), jnp.float32),
        pltpu.SemaphoreType.DMA,
    ],
    compiler_params=pltpu.CompilerParams(
        needs_layout_passes=False, use_tc_tiling_on_sc=False),
)
def _sc_main(fv_hbm, w_hbm, f_hbm, b_hbm, out_hbm,
             fv_v, w_v, f_v, b_v, out_v, fsem):
    wid = lax.axis_index("s") * NC + lax.axis_index("c")
    base = wid * BPW
    cps = [None, None]
    cps[0] = pltpu.async_copy(fv_hbm.at[pl.ds(base, CHR)], fv_v.at[0], fsem)
    pltpu.sync_copy(w_hbm, w_v)
    pltpu.sync_copy(f_hbm, f_v.at[pl.ds(0, 1), pl.ds(0, 1)])
    pltpu.sync_copy(b_hbm, b_v.at[pl.ds(0, 1)])
    lanes = lax.iota(jnp.int32, 16)
    ones16 = jnp.full((16,), 1.0, jnp.float32)
    fvec = ones16 * f_v[0, pl.ds(0, 16)][0]
    bias = ones16 * b_v[pl.ds(0, 16)][0]

    for k in range(NCHR):
        if k + 1 < NCHR:
            cps[(k + 1) % 2] = pltpu.async_copy(
                fv_hbm.at[pl.ds(base + (k + 1) * CHR, CHR)], fv_v.at[(k + 1) % 2], fsem)
        cps[k % 2].wait()
        fvk = fv_v.at[k % 2]

        def group(g, carry):
            r0 = g * 16
            p0 = k * CHR + r0
            out_v[0, pl.ds(p0, 16)] = ones16
            out_v[1, pl.ds(p0, 16)] = fvec
            dots = jnp.zeros((16,), jnp.float32)
            for r in range(16):
                acc = fvk[r0 + r, pl.ds(0, 16)] * w_v[0, pl.ds(0, 16)]
                for c in range(1, 8):
                    acc = acc + fvk[r0 + r, pl.ds(c * 16, 16)] * w_v[0, pl.ds(c * 16, 16)]
                dots = jnp.where(lanes == r, jnp.sum(acc), dots)
            out_v[3, pl.ds(p0, 16)] = dots + bias
            return carry

        lax.fori_loop(0, CHR // 16, group, 0)

    pltpu.sync_copy(out_v, out_hbm.at[:, pl.ds(base, BPW)])


@functools.partial(
    pl.kernel,
    out_type=jax.ShapeDtypeStruct((4, B), jnp.float32),
    mesh=_sc_mesh,
    scratch_types=[
        pltpu.VMEM((NCH, CHUNK), jnp.int32),
        pltpu.VMEM((NCH, CHUNK), jnp.float32),
        pltpu.VMEM((4, BPW), jnp.float32),
        pltpu.SemaphoreType.DMA,
    ],
    compiler_params=pltpu.CompilerParams(
        needs_layout_passes=False, use_tc_tiling_on_sc=False),
)
def _sc_gather(k1_hbm, tflat_hbm, idx_hbm, out_hbm, idx_v, e_v, out_v, gsem):
    wid = lax.axis_index("s") * NC + lax.axis_index("c")
    base = wid * BPW
    pltpu.sync_copy(idx_hbm.at[wid], idx_v)
    tflat = tflat_hbm.at[0]
    gathers = [
        pltpu.async_copy(tflat.at[idx_v.at[j]], e_v.at[j], gsem)
        for j in range(NCH)
    ]
    pltpu.sync_copy(k1_hbm.at[:, pl.ds(base, BPW)], out_v)
    for c in gathers:
        c.wait()
    lanes = lax.iota(jnp.int32, 16)

    def egroup(g, carry):
        rows = g * 16 + lanes
        ev = plsc.load_gather(e_v, [rows >> 7, rows & 127])
        out_v[2, pl.ds(g * 16, 16)] = ev
        return carry

    lax.fori_loop(0, BPW // 16, egroup, 0)
    pltpu.sync_copy(out_v, out_hbm.at[:, pl.ds(base, BPW)])


def kernel(fixed_vectors, item_id, F_param, emb_table, T_weight, T_bias):
    idx = jnp.asarray(item_id, jnp.int32).reshape(NW, NCH, CHUNK)
    k1_out = _sc_main(fixed_vectors, T_weight, F_param, T_bias)
    out_t = _sc_gather(k1_out, emb_table.T, idx)
    return out_t.T


# speedup vs baseline: 1.1987x; 1.0006x over previous
import functools

import jax
import jax.numpy as jnp
from jax import lax
from jax.experimental import pallas as pl
from jax.experimental.pallas import tpu as pltpu
from jax.experimental.pallas import tpu_sc as plsc

B = 16384
D = 128
V = 1000000
NC = 2
NS = 16
NW = NC * NS
BPW = B // NW
CHUNK = 128
NCH = BPW // CHUNK
CHR = 128
NCHR = BPW // CHR

_sc_mesh = plsc.VectorSubcoreMesh(core_axis_name="c", subcore_axis_name="s")


@functools.partial(
    pl.kernel,
    out_type=jax.ShapeDtypeStruct((4, B), jnp.float32),
    mesh=_sc_mesh,
    scratch_types=[
        pltpu.VMEM((2, CHR, D), jnp.float32),
        pltpu.VMEM((1, D), jnp.float32),
        pltpu.VMEM((1, 16), jnp.float32),
        pltpu.VMEM((16,), jnp.float32),
        pltpu.VMEM((4, BPW), jnp.float32),
        pltpu.SemaphoreType.DMA,
    ],
    compiler_params=pltpu.CompilerParams(
        needs_layout_passes=False, use_tc_tiling_on_sc=False),
)
def _sc_main(fv_hbm, w_hbm, f_hbm, b_hbm, out_hbm,
             fv_v, w_v, f_v, b_v, out_v, fsem):
    wid = lax.axis_index("s") * NC + lax.axis_index("c")
    base = wid * BPW
    cps = [None, None]
    cps[0] = pltpu.async_copy(fv_hbm.at[pl.ds(base, CHR)], fv_v.at[0], fsem)
    pltpu.sync_copy(w_hbm, w_v)
    pltpu.sync_copy(f_hbm, f_v.at[pl.ds(0, 1), pl.ds(0, 1)])
    pltpu.sync_copy(b_hbm, b_v.at[pl.ds(0, 1)])
    lanes = lax.iota(jnp.int32, 16)
    ones16 = jnp.full((16,), 1.0, jnp.float32)
    fvec = ones16 * f_v[0, pl.ds(0, 16)][0]
    bias = ones16 * b_v[pl.ds(0, 16)][0]

    for k in range(NCHR):
        if k + 1 < NCHR:
            cps[(k + 1) % 2] = pltpu.async_copy(
                fv_hbm.at[pl.ds(base + (k + 1) * CHR, CHR)], fv_v.at[(k + 1) % 2], fsem)
        cps[k % 2].wait()
        fvk = fv_v.at[k % 2]

        def group(g, carry):
            r0 = g * 16
            p0 = k * CHR + r0
            out_v[0, pl.ds(p0, 16)] = ones16
            out_v[1, pl.ds(p0, 16)] = fvec
            dots = jnp.zeros((16,), jnp.float32)
            for r in range(16):
                acc = fvk[r0 + r, pl.ds(0, 16)] * w_v[0, pl.ds(0, 16)]
                for c in range(1, 8):
                    acc = acc + fvk[r0 + r, pl.ds(c * 16, 16)] * w_v[0, pl.ds(c * 16, 16)]
                dots = jnp.where(lanes == r, jnp.sum(acc), dots)
            out_v[3, pl.ds(p0, 16)] = dots + bias
            return carry

        lax.fori_loop(0, CHR // 16, group, 0)

    pltpu.sync_copy(out_v, out_hbm.at[:, pl.ds(base, BPW)])


@functools.partial(
    pl.kernel,
    out_type=jax.ShapeDtypeStruct((4, B), jnp.float32),
    mesh=_sc_mesh,
    scratch_types=[
        pltpu.VMEM((NCH, CHUNK), jnp.int32),
        pltpu.VMEM((NCH, CHUNK), jnp.float32),
        pltpu.VMEM((4, BPW), jnp.float32),
        pltpu.SemaphoreType.DMA,
    ],
    compiler_params=pltpu.CompilerParams(
        needs_layout_passes=False, use_tc_tiling_on_sc=False),
)
def _sc_gather(k1_hbm, tflat_hbm, idx_hbm, out_hbm, idx_v, e_v, out_v, gsem):
    wid = lax.axis_index("s") * NC + lax.axis_index("c")
    base = wid * BPW
    pltpu.sync_copy(idx_hbm.at[wid], idx_v)
    tflat = tflat_hbm.at[0]
    gathers = [
        pltpu.async_copy(tflat.at[idx_v.at[j]], e_v.at[j], gsem)
        for j in range(NCH)
    ]
    pltpu.sync_copy(k1_hbm.at[:, pl.ds(base, BPW)], out_v)
    for c in gathers:
        c.wait()
    lanes = lax.iota(jnp.int32, 16)

    def egroup(g, carry):
        rows = g * 16 + lanes
        ev = plsc.load_gather(e_v, [rows >> 7, rows & 127])
        out_v[2, pl.ds(g * 16, 16)] = ev
        return carry

    lax.fori_loop(0, BPW // 16, egroup, 0)
    pltpu.sync_copy(out_v, out_hbm.at[:, pl.ds(base, BPW)])


def kernel(fixed_vectors, item_id, F_param, emb_table, T_weight, T_bias):
    idx = jnp.asarray(item_id, jnp.int32).reshape(NW, NCH, CHUNK)
    k1_out = _sc_main(fixed_vectors, T_weight, F_param, T_bias)
    out_t = _sc_gather(k1_out, emb_table.T, idx)
    return out_t.T
